# baseline (speedup 1.0000x reference)
import jax
import jax.numpy as jnp
from jax import lax
from jax.experimental import pallas as pl
from jax.experimental.pallas import tpu as pltpu
from jax.experimental.pallas import tpu_sc as plsc

B, L = 4096, 50
N_TOK = B * L
D_SOIL, D_CROP, D_WEATHER = 128, 64, 32
D_FEAT = D_SOIL + D_CROP + D_WEATHER
VOCAB = 1000
VPAD = 1024
NCH = 8

NC, NS, LANES = 2, 16, 16
NW = NC * NS
CHUNK = N_TOK // NW
GROUPS = CHUNK // LANES


def _project_body(es_ref, ec_ref, ew_ref, w_ref, b_ref, out_ref):
    w = w_ref[:]
    out_ref[0:VPAD, :] = (
        jnp.dot(es_ref[:], w[0:D_SOIL, :], preferred_element_type=jnp.float32)
        + b_ref[:]
    )
    out_ref[VPAD:2 * VPAD, :] = jnp.dot(
        ec_ref[:], w[D_SOIL:D_SOIL + D_CROP, :],
        preferred_element_type=jnp.float32)
    out_ref[2 * VPAD:3 * VPAD, :] = jnp.dot(
        ew_ref[:], w[D_SOIL + D_CROP:D_FEAT, :],
        preferred_element_type=jnp.float32)


_project = pl.pallas_call(
    _project_body,
    out_shape=jax.ShapeDtypeStruct((3 * VPAD, NCH), jnp.float32),
)


LPAD = 128
BPW = B // NW
BPH = BPW // 2
TPH = BPH * L
GPH = TPH // LANES


def _sc_body(p_hbm, s_hbm, c_hbm, w_hbm, o_hbm,
             p_v, s_v, c_v, w_v, o_v):
    wid = lax.axis_index("s") * NC + lax.axis_index("c")
    base = wid * CHUNK
    base_b = wid * BPW
    pltpu.sync_copy(p_hbm, p_v)
    pltpu.sync_copy(s_hbm.at[pl.ds(base, CHUNK)], s_v)
    pltpu.sync_copy(c_hbm.at[pl.ds(base, CHUNK)], c_v)
    pltpu.sync_copy(w_hbm.at[pl.ds(base, CHUNK)], w_v)

    iota = lax.iota(jnp.int32, LANES)

    for h in range(2):
        def body(g, carry):
            b0, l0 = carry
            off = h * TPH + g * LANES
            s8 = s_v[pl.ds(off, LANES)] * NCH
            c8 = c_v[pl.ds(off, LANES)] * NCH + VPAD * NCH
            w8 = w_v[pl.ds(off, LANES)] * NCH + 2 * VPAD * NCH
            ch = []
            for c in range(7):
                ch.append(plsc.load_gather(p_v, [s8 + c])
                          + plsc.load_gather(p_v, [c8 + c])
                          + plsc.load_gather(p_v, [w8 + c]))
            l = l0 + iota
            wrap = (l >= L).astype(jnp.int32)
            dest = (b0 + wrap) * LPAD + (l - L * wrap)
            for c in range(7):
                plsc.store_scatter(o_v, [dest + c * (BPH * LPAD)], ch[c])
            l0n = l0 + LANES
            swrap = (l0n >= L).astype(jnp.int32)
            return (b0 + swrap, l0n - L * swrap)

        lax.fori_loop(0, GPH, body, (jnp.int32(0), jnp.int32(0)))
        for c in range(7):
            pltpu.sync_copy(
                o_v.at[pl.ds(c * (BPH * LPAD), BPH * LPAD)],
                o_hbm.at[pl.ds((c * B + base_b + h * BPH) * LPAD,
                               BPH * LPAD)])


_sc_lookup = pl.kernel(
    _sc_body,
    out_type=jax.ShapeDtypeStruct((7 * B * LPAD,), jnp.float32),
    mesh=plsc.VectorSubcoreMesh(core_axis_name="c", subcore_axis_name="s"),
    compiler_params=pltpu.CompilerParams(needs_layout_passes=False),
    scratch_types=[
        pltpu.VMEM((3 * VPAD * NCH,), jnp.float32),
        pltpu.VMEM((CHUNK,), jnp.int32),
        pltpu.VMEM((CHUNK,), jnp.int32),
        pltpu.VMEM((CHUNK,), jnp.int32),
        pltpu.VMEM((7 * BPH * LPAD,), jnp.float32),
    ],
)


TB = 128
RB = TB * L // 128
NBLK = B // TB


def _finish_body(p_ref, y_ref, a_ref):
    def tok(c):
        return p_ref[c, 0, :, 0:L]
    y_ref[...] = tok(0).reshape(TB, L, 1)
    ls = [tok(c) for c in range(1, 7)]
    m = ls[0]
    for x in ls[1:]:
        m = jnp.maximum(m, x)
    es = [jnp.exp(x - m) for x in ls]
    s = es[0]
    for e in es[1:]:
        s = s + e
    inv = 1.0 / s
    a_ref[...] = jnp.concatenate(
        [(e * inv)[..., None] for e in es], axis=-1)


_finish = pl.pallas_call(
    _finish_body,
    grid=(NBLK,),
    in_specs=[pl.BlockSpec((7, 1, TB, LPAD), lambda i: (0, i, 0, 0))],
    out_specs=[pl.BlockSpec((TB, L, 1), lambda i: (i, 0, 0)),
               pl.BlockSpec((TB, L, 6), lambda i: (i, 0, 0))],
    out_shape=[jax.ShapeDtypeStruct((B, L, 1), jnp.float32),
               jax.ShapeDtypeStruct((B, L, 6), jnp.float32)],
)


def kernel(soil_idx, crop_idx, weather_idx, E_soil, E_crop, E_weather,
           W_yield, b_yield, W_alloc, b_alloc):
    f32 = jnp.float32
    es = jnp.pad(E_soil, ((0, VPAD - VOCAB), (0, 0)))
    ec = jnp.pad(E_crop, ((0, VPAD - VOCAB), (0, 0)))
    ew = jnp.pad(E_weather, ((0, VPAD - VOCAB), (0, 0)))
    wcat = jnp.concatenate(
        [W_yield, W_alloc, jnp.zeros((D_FEAT, 1), f32)], axis=1)
    bcat = jnp.concatenate(
        [b_yield, b_alloc, jnp.zeros((1,), f32)]).reshape(1, NCH)
    p = _project(es, ec, ew, wcat, bcat).reshape(-1)
    si = soil_idx.reshape(-1).astype(jnp.int32)
    ci = crop_idx.reshape(-1).astype(jnp.int32)
    wi = weather_idx.reshape(-1).astype(jnp.int32)
    planes = _sc_lookup(p, si, ci, wi).reshape(7, NBLK, TB, LPAD)
    y, a = _finish(planes)
    return y, a


# speedup vs baseline: 3.2620x; 3.2620x over previous
import jax
import jax.numpy as jnp
from jax import lax
from jax.experimental import pallas as pl
from jax.experimental.pallas import tpu as pltpu
from jax.experimental.pallas import tpu_sc as plsc

B, L = 4096, 50
N_TOK = B * L
D_SOIL, D_CROP, D_WEATHER = 128, 64, 32
D_FEAT = D_SOIL + D_CROP + D_WEATHER
VOCAB = 1000
VPAD = 1024
NCH = 8

NC, NS, LANES = 2, 16, 16
NW = NC * NS
CHUNK = N_TOK // NW
GROUPS = CHUNK // LANES


def _project_body(es_ref, ec_ref, ew_ref, w_ref, b_ref, out_ref):
    w = w_ref[:]
    out_ref[0:VPAD, :] = (
        jnp.dot(es_ref[:], w[0:D_SOIL, :], preferred_element_type=jnp.float32)
        + b_ref[:]
    )
    out_ref[VPAD:2 * VPAD, :] = jnp.dot(
        ec_ref[:], w[D_SOIL:D_SOIL + D_CROP, :],
        preferred_element_type=jnp.float32)
    out_ref[2 * VPAD:3 * VPAD, :] = jnp.dot(
        ew_ref[:], w[D_SOIL + D_CROP:D_FEAT, :],
        preferred_element_type=jnp.float32)


_project = pl.pallas_call(
    _project_body,
    out_shape=jax.ShapeDtypeStruct((3 * VPAD, NCH), jnp.float32),
)


def _sc_body(p_hbm, s_hbm, c_hbm, w_hbm, o_hbm,
             p_v, s_v, c_v, w_v, o_v):
    wid = lax.axis_index("s") * NC + lax.axis_index("c")
    base = wid * CHUNK
    pltpu.sync_copy(p_hbm, p_v)
    pltpu.sync_copy(s_hbm.at[pl.ds(base, CHUNK)], s_v)
    pltpu.sync_copy(c_hbm.at[pl.ds(base, CHUNK)], c_v)
    pltpu.sync_copy(w_hbm.at[pl.ds(base, CHUNK)], w_v)

    @plsc.parallel_loop(0, GROUPS, 1, unroll=4)
    def _loop(g):
        off = g * LANES
        s8 = s_v[pl.ds(off, LANES)] * NCH
        c8 = c_v[pl.ds(off, LANES)] * NCH + VPAD * NCH
        w8 = w_v[pl.ds(off, LANES)] * NCH + 2 * VPAD * NCH
        ch = []
        for c in range(7):
            ch.append(plsc.load_gather(p_v, [s8 + c])
                      + plsc.load_gather(p_v, [c8 + c])
                      + plsc.load_gather(p_v, [w8 + c]))
        m = ch[1]
        for c in range(2, 7):
            m = jnp.maximum(m, ch[c])
        es = [jnp.exp(ch[c] - m) for c in range(1, 7)]
        tot = es[0]
        for e in es[1:]:
            tot = tot + e
        inv = 1.0 / tot
        o_v[pl.ds(off, LANES)] = ch[0]
        for j in range(6):
            o_v[pl.ds((j + 1) * CHUNK + off, LANES)] = es[j] * inv

    for c in range(7):
        pltpu.sync_copy(o_v.at[pl.ds(c * CHUNK, CHUNK)],
                        o_hbm.at[pl.ds(c * N_TOK + base, CHUNK)])


_sc_lookup = pl.kernel(
    _sc_body,
    out_type=jax.ShapeDtypeStruct((7 * N_TOK,), jnp.float32),
    mesh=plsc.VectorSubcoreMesh(core_axis_name="c", subcore_axis_name="s"),
    compiler_params=pltpu.CompilerParams(needs_layout_passes=False),
    scratch_types=[
        pltpu.VMEM((3 * VPAD * NCH,), jnp.float32),
        pltpu.VMEM((CHUNK,), jnp.int32),
        pltpu.VMEM((CHUNK,), jnp.int32),
        pltpu.VMEM((CHUNK,), jnp.int32),
        pltpu.VMEM((7 * CHUNK,), jnp.float32),
    ],
)


def kernel(soil_idx, crop_idx, weather_idx, E_soil, E_crop, E_weather,
           W_yield, b_yield, W_alloc, b_alloc):
    f32 = jnp.float32
    es = jnp.pad(E_soil, ((0, VPAD - VOCAB), (0, 0)))
    ec = jnp.pad(E_crop, ((0, VPAD - VOCAB), (0, 0)))
    ew = jnp.pad(E_weather, ((0, VPAD - VOCAB), (0, 0)))
    wcat = jnp.concatenate(
        [W_yield, W_alloc, jnp.zeros((D_FEAT, 1), f32)], axis=1)
    bcat = jnp.concatenate(
        [b_yield, b_alloc, jnp.zeros((1,), f32)]).reshape(1, NCH)
    p = _project(es, ec, ew, wcat, bcat).reshape(-1)
    si = soil_idx.reshape(-1).astype(jnp.int32)
    ci = crop_idx.reshape(-1).astype(jnp.int32)
    wi = weather_idx.reshape(-1).astype(jnp.int32)
    planes = _sc_lookup(p, si, ci, wi).reshape(7, B, L)
    y = planes[0].reshape(B, L, 1)
    a = jnp.einsum('dbl,dc->blc', planes[1:7], jnp.eye(6, dtype=f32))
    return y, a
